# baseline (speedup 1.0000x reference)
from functools import partial

import jax
import jax.numpy as jnp
from jax.experimental import pallas as pl
from jax.experimental.pallas import tpu as pltpu

_EPS = 1e-6


def _gem_kernel(p_ref, x_ref, o_ref, *, inv_hw):
    p = p_ref[0]
    xc = jnp.maximum(x_ref[...], _EPS)
    t = jnp.exp2(p * jnp.log2(xc))
    s_col = jnp.sum(t, axis=-1, keepdims=True)
    s_row = jnp.transpose(s_col) * inv_hw
    o_ref[...] = jnp.exp2(jnp.log2(s_row) / p)[None]


def _gem(x2, p_arr, br):
    rows, hw = x2.shape
    grid = rows // br
    return pl.pallas_call(
        partial(_gem_kernel, inv_hw=1.0 / hw),
        out_shape=jax.ShapeDtypeStruct((grid, 1, br), jnp.float32),
        grid=(grid,),
        in_specs=[
            pl.BlockSpec(memory_space=pltpu.MemorySpace.SMEM),
            pl.BlockSpec((br, hw), lambda i: (i, 0)),
        ],
        out_specs=pl.BlockSpec((1, 1, br), lambda i: (i, 0, 0)),
        compiler_params=pltpu.CompilerParams(
            dimension_semantics=("parallel",),
        ),
        cost_estimate=pl.CostEstimate(
            flops=5 * rows * hw,
            transcendentals=2 * rows * hw,
            bytes_accessed=rows * hw * 4 + rows * 4,
        ),
    )(p_arr, x2)


def kernel(x, p):
    N, C, H, W = x.shape
    hw = H * W
    rows = N * C
    x2 = x.reshape(rows, hw).astype(jnp.float32)
    p_arr = jnp.asarray(p, jnp.float32).reshape(1)

    br = 4096
    while rows % br:
        br //= 2

    out = _gem(x2, p_arr, br)
    return out.reshape(N, C, 1, 1).astype(x.dtype)


# speedup vs baseline: 1.8656x; 1.8656x over previous
from functools import partial

import jax
import jax.numpy as jnp
from jax.experimental import pallas as pl
from jax.experimental.pallas import tpu as pltpu

_EPS = 1e-6
_STREAMS = 2


def _gem_kernel(p_ref, *refs, inv_hw):
    x_refs, o_ref = refs[:-1], refs[-1]
    p = p_ref[0]
    cs = o_ref.shape[1] // len(x_refs)
    for i, x_ref in enumerate(x_refs):
        xc = jnp.maximum(x_ref[...], _EPS)
        t = jnp.exp2(p * jnp.log2(xc))
        s = jnp.sum(t, axis=-1, keepdims=True) * inv_hw
        o_ref[:, i * cs:(i + 1) * cs, :] = jnp.exp2(jnp.log2(s) / p)


def _gem(x3, p_arr, nb, streams):
    N, C, hw = x3.shape
    cs = C // streams
    grid = N // nb

    def _mk_spec(k):
        return pl.BlockSpec((nb, cs, hw), lambda i, _k=k: (i, _k, 0))

    return pl.pallas_call(
        partial(_gem_kernel, inv_hw=1.0 / hw),
        out_shape=jax.ShapeDtypeStruct((N, C, 1), jnp.float32),
        grid=(grid,),
        in_specs=[pl.BlockSpec(memory_space=pltpu.MemorySpace.SMEM)]
        + [_mk_spec(k) for k in range(streams)],
        out_specs=pl.BlockSpec((nb, C, 1), lambda i: (i, 0, 0)),
        compiler_params=pltpu.CompilerParams(
            dimension_semantics=("arbitrary",),
        ),
        cost_estimate=pl.CostEstimate(
            flops=5 * N * C * hw,
            transcendentals=2 * N * C * hw,
            bytes_accessed=N * C * hw * 4 + N * C * 4,
        ),
    )(p_arr, *([x3] * streams))


def kernel(x, p):
    N, C, H, W = x.shape
    hw = H * W
    x3 = x.reshape(N, C, hw).astype(jnp.float32)
    p_arr = jnp.asarray(p, jnp.float32).reshape(1)

    nb = 16
    while N % nb:
        nb //= 2

    out = _gem(x3, p_arr, nb, _STREAMS)
    return out.reshape(N, C, 1, 1).astype(x.dtype)


# speedup vs baseline: 5.8903x; 3.1573x over previous
from functools import partial

import jax
import jax.numpy as jnp
from jax.experimental import pallas as pl
from jax.experimental.pallas import tpu as pltpu

_EPS = 1e-6


def _gem_kernel(p_ref, x_ref, o_ref, *, inv_hw):
    p = p_ref[0]
    xc = jnp.maximum(x_ref[...], _EPS)
    t = jnp.exp2(p * jnp.log2(xc))
    s = jnp.sum(t, axis=1) * inv_hw
    o_ref[...] = jnp.exp2(jnp.log2(s) / p)


def _gem(xt, p_arr, nb):
    N, hw, C = xt.shape
    grid = N // nb
    return pl.pallas_call(
        partial(_gem_kernel, inv_hw=1.0 / hw),
        out_shape=jax.ShapeDtypeStruct((N, C), jnp.float32),
        grid=(grid,),
        in_specs=[
            pl.BlockSpec(memory_space=pltpu.MemorySpace.SMEM),
            pl.BlockSpec((nb, hw, C), lambda i: (i, 0, 0)),
        ],
        out_specs=pl.BlockSpec((nb, C), lambda i: (i, 0)),
        compiler_params=pltpu.CompilerParams(
            dimension_semantics=("arbitrary",),
        ),
        cost_estimate=pl.CostEstimate(
            flops=5 * N * C * hw,
            transcendentals=2 * N * C * hw,
            bytes_accessed=N * C * hw * 4 + N * C * 4,
        ),
    )(p_arr, xt)


def kernel(x, p):
    N, C, H, W = x.shape
    hw = H * W
    xt = jnp.transpose(x, (0, 2, 3, 1)).reshape(N, hw, C).astype(jnp.float32)
    p_arr = jnp.asarray(p, jnp.float32).reshape(1)

    nb = 8
    while N % nb:
        nb //= 2

    out = _gem(xt, p_arr, nb)
    return out.reshape(N, C, 1, 1).astype(x.dtype)


# speedup vs baseline: 6.3198x; 1.0729x over previous
from functools import partial

import jax
import jax.numpy as jnp
from jax.experimental import pallas as pl
from jax.experimental.pallas import tpu as pltpu

_EPS = 1e-6


def _gem_kernel(p_ref, x_ref, o_ref, *, inv_hw):
    p = p_ref[0]
    xc = jnp.maximum(x_ref[...], _EPS)
    t = jnp.exp2(p * jnp.log2(xc))
    s = jnp.sum(t, axis=1) * inv_hw
    o_ref[...] = jnp.exp2(jnp.log2(s) / p)


def _gem(xt, p_arr, nb):
    N, hw, C = xt.shape
    grid = N // nb
    return pl.pallas_call(
        partial(_gem_kernel, inv_hw=1.0 / hw),
        out_shape=jax.ShapeDtypeStruct((N, C), jnp.float32),
        grid=(grid,),
        in_specs=[
            pl.BlockSpec(memory_space=pltpu.MemorySpace.SMEM),
            pl.BlockSpec((nb, hw, C), lambda i: (i, 0, 0)),
        ],
        out_specs=pl.BlockSpec((nb, C), lambda i: (i, 0)),
        compiler_params=pltpu.CompilerParams(
            dimension_semantics=("arbitrary",),
        ),
        cost_estimate=pl.CostEstimate(
            flops=5 * N * C * hw,
            transcendentals=2 * N * C * hw,
            bytes_accessed=N * C * hw * 4 + N * C * 4,
        ),
    )(p_arr, xt)


def kernel(x, p):
    N, C, H, W = x.shape
    hw = H * W
    xt = jnp.transpose(x, (0, 2, 3, 1)).reshape(N, hw, C).astype(jnp.float32)
    p_arr = jnp.asarray(p, jnp.float32).reshape(1)

    nb = 16
    while N % nb:
        nb //= 2

    out = _gem(xt, p_arr, nb)
    return out.reshape(N, C, 1, 1).astype(x.dtype)
